# baseline (speedup 1.0000x reference)
import functools

import jax
import jax.numpy as jnp
from jax import lax
from jax.experimental import pallas as pl
from jax.experimental.pallas import tpu as pltpu
from jax.experimental.pallas import tpu_sc as plsc

HIS = 50
K = 5
N = HIS * K
D = 128
SL = 100
DK = K * D
VOCAB = 30522
TBL = 30528
GCOLS = 64
CLEN = HIS * GCOLS
NCHUNK = 16
BS = 4
NW = 32


def _lane_iota():
    return lax.broadcasted_iota(jnp.int32, (16,), 0)


def _take16(x, idx):
    dnums = lax.GatherDimensionNumbers(
        offset_dims=(), collapsed_slice_dims=(0,), start_index_map=(0,))
    return lax.gather(x, idx[:, None], dnums, (1,),
                      mode=lax.GatherScatterMode.PROMISE_IN_BOUNDS)


def _make_sc_body(ex_per_w):
    def _sc_body(his_hbm, ids_hbm, cnt_hbm, his_v, ids_v, vbuf, table, cflat):
        nc = 2
        wid = lax.axis_index("s") * nc + lax.axis_index("c")
        lanes = _lane_iota()
        zeros16 = jnp.zeros((16,), jnp.int32)
        zerosf = jnp.zeros((16,), jnp.float32)
        ones16 = jnp.ones((16,), jnp.float32)

        def _zt(k, _):
            table[pl.ds(k * 16, 16)] = zeros16
            return _
        lax.fori_loop(0, TBL // 16, _zt, 0)

        def _example(ex, _):
            i = wid * ex_per_w + ex
            pltpu.sync_copy(his_hbm.at[i], his_v)
            pltpu.sync_copy(ids_hbm.at[i], ids_v)

            def _zc(k, _c):
                cflat[pl.ds(k * 16, 16)] = zerosf
                return _c
            lax.fori_loop(0, CLEN // 16, _zc, 0)

            for c in range(NCHUNK):
                jvec = c * 16 + lanes
                hvec = jnp.minimum(jvec // K, HIS - 1)
                tvec = ids_v[pl.ds(c * 16, 16)]
                addr = hvec * SL + tvec
                vbuf[pl.ds(c * 16, 16)] = plsc.load_gather(his_v, [addr])

            cnt = jnp.int32(0)
            for c in range(NCHUNK):
                jvec = c * 16 + lanes
                lanemask = jvec < N
                vvec = vbuf[pl.ds(c * 16, 16)]
                g16 = plsc.load_gather(table, [vvec])
                fpl = jnp.full((16,), 16, jnp.int32)
                for b in range(15, -1, -1):
                    bv = _take16(vvec, jnp.full((16,), b, jnp.int32))
                    fpl = jnp.where(vvec == bv, b, fpl)
                new_first = (fpl == lanes) & (g16 == 0) & lanemask
                rank = jnp.cumsum(new_first.astype(jnp.int32))
                gid_new = cnt + rank - 1
                gid16 = jnp.where(g16 > 0, g16 - 1, _take16(gid_new, fpl))
                plsc.store_scatter(table, [vvec], gid16 + 1, mask=lanemask)
                cnt = cnt + jnp.sum(new_first.astype(jnp.int32))
                cmask = lanemask & (gid16 < HIS)
                caddr = gid16 * GCOLS + ids_v[pl.ds(c * 16, 16)]
                plsc.addupdate_scatter(cflat, [caddr], ones16, mask=cmask)

            for c in range(NCHUNK):
                jvec = c * 16 + lanes
                vvec = vbuf[pl.ds(c * 16, 16)]
                plsc.store_scatter(table, [vvec], zeros16, mask=jvec < N)

            pltpu.sync_copy(cflat, cnt_hbm.at[i])
            return _
        lax.fori_loop(0, ex_per_w, _example, 0)
    return _sc_body


def _counts_kernel(his2d, ids2d):
    bh = his2d.shape[0]
    mesh = plsc.VectorSubcoreMesh(core_axis_name="c", subcore_axis_name="s")
    f = functools.partial(
        pl.kernel, mesh=mesh,
        compiler_params=pltpu.CompilerParams(needs_layout_passes=False),
        out_type=jax.ShapeDtypeStruct((bh, CLEN), jnp.float32),
        scratch_types=[
            pltpu.VMEM((HIS * SL,), jnp.int32),
            pltpu.VMEM((256,), jnp.int32),
            pltpu.VMEM((256,), jnp.int32),
            pltpu.VMEM((TBL,), jnp.int32),
            pltpu.VMEM((CLEN,), jnp.float32),
        ],
    )(_make_sc_body(bh // NW))
    return f(his2d, ids2d)


def _tc_body(cnt_ref, terms_ref, out_ref):
    jrow = lax.broadcasted_iota(jnp.int32, (N, HIS), 0)
    gcol = lax.broadcasted_iota(jnp.int32, (N, HIS), 1)
    e5rep = (jrow // K == gcol).astype(jnp.float32)
    kmask = [(jrow % K == k) for k in range(K)]
    for e in range(BS):
        counts = cnt_ref[e][:, :HIS]
        crep = jnp.dot(e5rep, counts, preferred_element_type=jnp.float32)
        t2 = terms_ref[e].reshape(HIS, DK)
        acc = None
        for k in range(K):
            ck = jnp.where(kmask[k], crep, 0.0)
            part = jnp.dot(ck, t2[:, k * D:(k + 1) * D],
                           preferred_element_type=jnp.float32)
            acc = part if acc is None else acc + part
        out_ref[e] = acc


def _dense(cnt_half, terms_half):
    bh = cnt_half.shape[0]
    return pl.pallas_call(
        _tc_body,
        grid=(bh // BS,),
        in_specs=[
            pl.BlockSpec((BS, HIS, GCOLS), lambda i: (i, 0, 0)),
            pl.BlockSpec((BS, HIS, K, D), lambda i: (i, 0, 0, 0)),
        ],
        out_specs=pl.BlockSpec((BS, N, D), lambda i: (i, 0, 0)),
        out_shape=jax.ShapeDtypeStruct((bh, N, D), jnp.float32),
    )(cnt_half, terms_half)


def kernel(ps_terms, ps_term_ids, his_news):
    B = ps_terms.shape[0]
    ids = ps_term_ids.reshape(B, N)
    ids_pad = jnp.pad(ids, ((0, 0), (0, 256 - N)))
    his2d = his_news.reshape(B, HIS * SL)

    h = B // 2
    cnt0 = _counts_kernel(his2d[:h], ids_pad[:h]).reshape(h, HIS, GCOLS)
    cnt1 = _counts_kernel(his2d[h:], ids_pad[h:]).reshape(h, HIS, GCOLS)
    out0 = _dense(cnt0, ps_terms[:h])
    out1 = _dense(cnt1, ps_terms[h:])
    return jnp.concatenate([out0, out1], axis=0)


# speedup vs baseline: 1.3391x; 1.3391x over previous
import functools

import jax
import jax.numpy as jnp
from jax import lax
from jax.experimental import pallas as pl
from jax.experimental.pallas import tpu as pltpu
from jax.experimental.pallas import tpu_sc as plsc

HIS = 50
K = 5
N = HIS * K
D = 128
SL = 100
DK = K * D
VOCAB = 30522
TBL = 30528
GCOLS = 64
CLEN = HIS * GCOLS
NCHUNK = 16
BS = 4
NW = 32


def _lane_iota():
    return lax.broadcasted_iota(jnp.int32, (16,), 0)


def _take16(x, idx):
    dnums = lax.GatherDimensionNumbers(
        offset_dims=(), collapsed_slice_dims=(0,), start_index_map=(0,))
    return lax.gather(x, idx[:, None], dnums, (1,),
                      mode=lax.GatherScatterMode.PROMISE_IN_BOUNDS)


def _make_sc_body(ex_per_w):
    def _sc_body(his_hbm, ids_hbm, cnt_hbm, his_v, ids_v, vbuf, table, cflat):
        nc = 2
        wid = lax.axis_index("s") * nc + lax.axis_index("c")
        lanes = _lane_iota()
        zeros16 = jnp.zeros((16,), jnp.int32)
        zerosf = jnp.zeros((16,), jnp.float32)
        ones16 = jnp.ones((16,), jnp.float32)

        def _zt(k, _):
            table[pl.ds(k * 16, 16)] = zeros16
            return _
        lax.fori_loop(0, TBL // 16, _zt, 0)

        def _example(ex, _):
            i = wid * ex_per_w + ex
            pltpu.sync_copy(his_hbm.at[i], his_v)
            pltpu.sync_copy(ids_hbm.at[i], ids_v)

            def _zc(k, _c):
                cflat[pl.ds(k * 16, 16)] = zerosf
                return _c
            lax.fori_loop(0, CLEN // 16, _zc, 0)

            for c in range(NCHUNK):
                jvec = c * 16 + lanes
                hvec = jnp.minimum(jvec // K, HIS - 1)
                tvec = ids_v[pl.ds(c * 16, 16)]
                addr = hvec * SL + tvec
                vbuf[pl.ds(c * 16, 16)] = plsc.load_gather(his_v, [addr])

            cnt = jnp.int32(0)
            for c in range(NCHUNK):
                jvec = c * 16 + lanes
                lanemask = jvec < N
                vvec = vbuf[pl.ds(c * 16, 16)]
                g16 = plsc.load_gather(table, [vvec])
                fpl = jnp.full((16,), 16, jnp.int32)
                for b in range(15, -1, -1):
                    bv = _take16(vvec, jnp.full((16,), b, jnp.int32))
                    fpl = jnp.where(vvec == bv, b, fpl)
                new_first = (fpl == lanes) & (g16 == 0) & lanemask
                rank = jnp.cumsum(new_first.astype(jnp.int32))
                gid_new = cnt + rank - 1
                gid16 = jnp.where(g16 > 0, g16 - 1, _take16(gid_new, fpl))
                plsc.store_scatter(table, [vvec], gid16 + 1, mask=lanemask)
                cnt = cnt + jnp.sum(new_first.astype(jnp.int32))
                cmask = lanemask & (gid16 < HIS)
                caddr = gid16 * GCOLS + ids_v[pl.ds(c * 16, 16)]
                plsc.addupdate_scatter(cflat, [caddr], ones16, mask=cmask)

            for c in range(NCHUNK):
                jvec = c * 16 + lanes
                vvec = vbuf[pl.ds(c * 16, 16)]
                plsc.store_scatter(table, [vvec], zeros16, mask=jvec < N)

            pltpu.sync_copy(cflat, cnt_hbm.at[i])
            return _
        lax.fori_loop(0, ex_per_w, _example, 0)
    return _sc_body


def _counts_kernel(his2d, ids2d):
    bh = his2d.shape[0]
    mesh = plsc.VectorSubcoreMesh(core_axis_name="c", subcore_axis_name="s")
    f = functools.partial(
        pl.kernel, mesh=mesh,
        compiler_params=pltpu.CompilerParams(needs_layout_passes=False),
        out_type=jax.ShapeDtypeStruct((bh, CLEN), jnp.float32),
        scratch_types=[
            pltpu.VMEM((HIS * SL,), jnp.int32),
            pltpu.VMEM((256,), jnp.int32),
            pltpu.VMEM((256,), jnp.int32),
            pltpu.VMEM((TBL,), jnp.int32),
            pltpu.VMEM((CLEN,), jnp.float32),
        ],
    )(_make_sc_body(bh // NW))
    return f(his2d, ids2d)


def _tc_body(cnt_ref, terms_ref, out_ref):
    jrow = lax.broadcasted_iota(jnp.int32, (N, HIS), 0)
    gcol = lax.broadcasted_iota(jnp.int32, (N, HIS), 1)
    e5rep = (jrow // K == gcol).astype(jnp.float32)
    kmask = [(jrow % K == k) for k in range(K)]
    for e in range(BS):
        counts = cnt_ref[e][:, :HIS]
        crep = jnp.dot(e5rep, counts, preferred_element_type=jnp.float32)
        t2 = terms_ref[e].reshape(HIS, DK)
        acc = None
        for k in range(K):
            ck = jnp.where(kmask[k], crep, 0.0)
            part = jnp.dot(ck, t2[:, k * D:(k + 1) * D],
                           preferred_element_type=jnp.float32)
            acc = part if acc is None else acc + part
        out_ref[e] = acc


def _dense(cnt_half, terms_half):
    bh = cnt_half.shape[0]
    return pl.pallas_call(
        _tc_body,
        grid=(bh // BS,),
        in_specs=[
            pl.BlockSpec((BS, HIS, GCOLS), lambda i: (i, 0, 0)),
            pl.BlockSpec((BS, HIS, K, D), lambda i: (i, 0, 0, 0)),
        ],
        out_specs=pl.BlockSpec((BS, N, D), lambda i: (i, 0, 0)),
        out_shape=jax.ShapeDtypeStruct((bh, N, D), jnp.float32),
    )(cnt_half, terms_half)


def kernel(ps_terms, ps_term_ids, his_news):
    B = ps_terms.shape[0]
    ids = ps_term_ids.reshape(B, N)
    ids_pad = jnp.pad(ids, ((0, 0), (0, 256 - N)))
    his2d = his_news.reshape(B, HIS * SL)

    counts = _counts_kernel(his2d, ids_pad).reshape(B, HIS, GCOLS)
    return _dense(counts, ps_terms)


# speedup vs baseline: 1.3941x; 1.0411x over previous
import functools

import jax
import jax.numpy as jnp
from jax import lax
from jax.experimental import pallas as pl
from jax.experimental.pallas import tpu as pltpu
from jax.experimental.pallas import tpu_sc as plsc

HIS = 50
K = 5
N = HIS * K
D = 128
SL = 100
DK = K * D
VOCAB = 30522
TBL = 30528
GCOLS = 56
CLEN = HIS * GCOLS
NCHUNK = 16
BS = 4
NW = 32


def _lane_iota():
    return lax.broadcasted_iota(jnp.int32, (16,), 0)


def _take16(x, idx):
    dnums = lax.GatherDimensionNumbers(
        offset_dims=(), collapsed_slice_dims=(0,), start_index_map=(0,))
    return lax.gather(x, idx[:, None], dnums, (1,),
                      mode=lax.GatherScatterMode.PROMISE_IN_BOUNDS)


def _make_sc_body(ex_per_w):
    def _sc_body(his_hbm, ids_hbm, cnt_hbm, his_v, ids_v, vbuf, table, cflat,
                 sem1, sem2):
        nc = 2
        wid = lax.axis_index("s") * nc + lax.axis_index("c")
        lanes = _lane_iota()
        zeros16 = jnp.zeros((16,), jnp.int32)
        zerosf = jnp.zeros((16,), jnp.float32)
        ones16 = jnp.ones((16,), jnp.float32)

        def _zt(k, _):
            table[pl.ds(k * 16, 16)] = zeros16
            return _
        lax.fori_loop(0, TBL // 16, _zt, 0)

        def _example(ex, _):
            i = wid * ex_per_w + ex
            cp_his = pltpu.async_copy(his_hbm.at[i], his_v, sem1)
            cp_ids = pltpu.async_copy(ids_hbm.at[i], ids_v, sem2)

            def _zc(k, _c):
                cflat[pl.ds(k * 16, 16)] = zerosf
                return _c
            lax.fori_loop(0, CLEN // 16, _zc, 0)
            cp_his.wait()
            cp_ids.wait()

            for c in range(NCHUNK):
                jvec = c * 16 + lanes
                hvec = jnp.minimum(jvec // K, HIS - 1)
                tvec = ids_v[pl.ds(c * 16, 16)]
                addr = hvec * SL + tvec
                vbuf[pl.ds(c * 16, 16)] = plsc.load_gather(his_v, [addr])

            cnt = jnp.int32(0)
            for c in range(NCHUNK):
                jvec = c * 16 + lanes
                lanemask = jvec < N
                vvec = vbuf[pl.ds(c * 16, 16)]
                g16 = plsc.load_gather(table, [vvec])
                cands = []
                for b in range(16):
                    bv = _take16(vvec, jnp.full((16,), b, jnp.int32))
                    cands.append(jnp.where(vvec == bv, b, 16))
                while len(cands) > 1:
                    cands = [jnp.minimum(a, bb)
                             for a, bb in zip(cands[::2], cands[1::2])]
                fpl = cands[0]
                new_first = (fpl == lanes) & (g16 == 0) & lanemask
                rank = jnp.cumsum(new_first.astype(jnp.int32))
                gid_new = cnt + rank - 1
                gid16 = jnp.where(g16 > 0, g16 - 1, _take16(gid_new, fpl))
                plsc.store_scatter(table, [vvec], gid16 + 1, mask=lanemask)
                cnt = cnt + jnp.sum(new_first.astype(jnp.int32))
                cmask = lanemask & (gid16 < HIS)
                caddr = gid16 * GCOLS + ids_v[pl.ds(c * 16, 16)]
                plsc.addupdate_scatter(cflat, [caddr], ones16, mask=cmask)

            for c in range(NCHUNK):
                jvec = c * 16 + lanes
                vvec = vbuf[pl.ds(c * 16, 16)]
                plsc.store_scatter(table, [vvec], zeros16, mask=jvec < N)

            pltpu.sync_copy(cflat, cnt_hbm.at[i])
            return _
        lax.fori_loop(0, ex_per_w, _example, 0)
    return _sc_body


def _counts_kernel(his2d, ids2d):
    bh = his2d.shape[0]
    mesh = plsc.VectorSubcoreMesh(core_axis_name="c", subcore_axis_name="s")
    f = functools.partial(
        pl.kernel, mesh=mesh,
        compiler_params=pltpu.CompilerParams(needs_layout_passes=False),
        out_type=jax.ShapeDtypeStruct((bh, CLEN), jnp.float32),
        scratch_types=[
            pltpu.VMEM((HIS * SL,), jnp.int32),
            pltpu.VMEM((256,), jnp.int32),
            pltpu.VMEM((256,), jnp.int32),
            pltpu.VMEM((TBL,), jnp.int32),
            pltpu.VMEM((CLEN,), jnp.float32),
            pltpu.SemaphoreType.DMA,
            pltpu.SemaphoreType.DMA,
        ],
    )(_make_sc_body(bh // NW))
    return f(his2d, ids2d)


def _tc_body(cnt_ref, terms_ref, out_ref):
    jrow = lax.broadcasted_iota(jnp.int32, (N, HIS), 0)
    gcol = lax.broadcasted_iota(jnp.int32, (N, HIS), 1)
    e5rep = (jrow // K == gcol).astype(jnp.float32)
    kmask = [(jrow % K == k) for k in range(K)]
    for e in range(BS):
        counts = cnt_ref[e][:, :HIS]
        crep = jnp.dot(e5rep, counts, preferred_element_type=jnp.float32)
        t2 = terms_ref[e].reshape(HIS, DK)
        acc = None
        for k in range(K):
            ck = jnp.where(kmask[k], crep, 0.0)
            part = jnp.dot(ck, t2[:, k * D:(k + 1) * D],
                           preferred_element_type=jnp.float32)
            acc = part if acc is None else acc + part
        out_ref[e] = acc


def _dense(cnt_half, terms_half):
    bh = cnt_half.shape[0]
    return pl.pallas_call(
        _tc_body,
        grid=(bh // BS,),
        in_specs=[
            pl.BlockSpec((BS, HIS, GCOLS), lambda i: (i, 0, 0)),
            pl.BlockSpec((BS, HIS, K, D), lambda i: (i, 0, 0, 0)),
        ],
        out_specs=pl.BlockSpec((BS, N, D), lambda i: (i, 0, 0)),
        out_shape=jax.ShapeDtypeStruct((bh, N, D), jnp.float32),
    )(cnt_half, terms_half)


def kernel(ps_terms, ps_term_ids, his_news):
    B = ps_terms.shape[0]
    ids = ps_term_ids.reshape(B, N)
    ids_pad = jnp.pad(ids, ((0, 0), (0, 256 - N)))
    his2d = his_news.reshape(B, HIS * SL)

    counts = _counts_kernel(his2d, ids_pad).reshape(B, HIS, GCOLS)
    return _dense(counts, ps_terms)
